# baseline (speedup 1.0000x reference)
import functools

import jax
import jax.numpy as jnp
from jax import lax
from jax.experimental import pallas as pl
from jax.experimental.pallas import tpu as pltpu
from jax.experimental.pallas import tpu_sc as plsc

N_NODES = 10000
N_EDGES = 320000
NW = 32

_MESH = dict(core_axis_name="c", subcore_axis_name="s")


def _wid():
    return lax.axis_index("s") * 2 + lax.axis_index("c")


def _sc_gather(in_vc, src, dst):
    CH = 80
    per_w = N_EDGES // NW

    @functools.partial(
        pl.kernel,
        mesh=plsc.VectorSubcoreMesh(**_MESH),
        compiler_params=pltpu.CompilerParams(needs_layout_passes=False),
        out_type=[
            jax.ShapeDtypeStruct((N_EDGES, 128), jnp.float32),
            jax.ShapeDtypeStruct((N_EDGES, 128), jnp.float32),
        ],
        scratch_types=[
            pltpu.VMEM((CH,), jnp.int32),
            pltpu.VMEM((CH, 128), jnp.float32),
            pltpu.SemaphoreType.DMA,
        ],
    )
    def k(vc_hbm, src_hbm, dst_hbm, gs_hbm, gd_hbm, idx_v, rows_v, sem):
        w = _wid()
        base0 = w * per_w

        def step(i, _):
            base = base0 + i * CH
            pltpu.sync_copy(src_hbm.at[pl.ds(base, CH)], idx_v)
            pltpu.async_copy(vc_hbm.at[idx_v], rows_v, sem).wait()
            pltpu.sync_copy(rows_v, gs_hbm.at[pl.ds(base, CH)])
            pltpu.sync_copy(dst_hbm.at[pl.ds(base, CH)], idx_v)
            pltpu.async_copy(vc_hbm.at[idx_v], rows_v, sem).wait()
            pltpu.sync_copy(rows_v, gd_hbm.at[pl.ds(base, CH)])
            return ()

        lax.fori_loop(0, per_w // CH, step, ())

    return k(in_vc, src, dst)


def _edge_mlp_body(gs, gd, ve, w1s, w1d, w1e, b1, w2f, b2f, w2k, b2k,
                   wvf, wve, bv, f1o, f4o, fmmo, oveo):
    f32 = jnp.float32
    gs16 = gs[...].astype(jnp.bfloat16)
    gd16 = gd[...].astype(jnp.bfloat16)
    ve16 = ve[...].astype(jnp.bfloat16)
    h1 = (jnp.dot(gs16, w1s[...], preferred_element_type=f32)
          + jnp.dot(gd16, w1d[...], preferred_element_type=f32)
          + jnp.dot(ve16, w1e[...], preferred_element_type=f32) + b1[...])
    h1 = jnp.maximum(h1, 0.0)
    h1b = h1.astype(jnp.bfloat16)
    hf = jnp.dot(h1b, w2f[...], preferred_element_type=f32) + b2f[...]
    hk = jnp.sum(h1 * w2k[...], axis=1, keepdims=True) + b2k[...]
    kgate = jax.nn.sigmoid(hk)
    f = hf * kgate
    f1o[...] = f[:, :128]
    f4o[...] = f[:, 384:]
    fmmo[...] = f[:, 128:384]
    oveo[...] = (jnp.dot(f.astype(jnp.bfloat16), wvf[...], preferred_element_type=f32)
                 + jnp.dot(ve16, wve[...], preferred_element_type=f32) + bv[...])


def _edge_mlp(gs, gd, ve, w1s, w1d, w1e, b1, w2f, b2f, w2k, b2k, wvf, wve, bv,
              interpret=False):
    EB = 2560
    grid = (N_EDGES // EB,)
    ebs = lambda cols: pl.BlockSpec((EB, cols), lambda i: (i, 0))
    full = lambda a: pl.BlockSpec(a.shape, lambda i: (0,) * a.ndim)
    return pl.pallas_call(
        _edge_mlp_body,
        grid=grid,
        in_specs=[ebs(128), ebs(128), ebs(128), full(w1s), full(w1d),
                  full(w1e), full(b1), full(w2f), full(b2f), full(w2k),
                  full(b2k), full(wvf), full(wve), full(bv)],
        out_specs=[ebs(128), ebs(128), ebs(256), ebs(128)],
        out_shape=[
            jax.ShapeDtypeStruct((N_EDGES, 128), jnp.float32),
            jax.ShapeDtypeStruct((N_EDGES, 128), jnp.float32),
            jax.ShapeDtypeStruct((N_EDGES, 256), jnp.float32),
            jax.ShapeDtypeStruct((N_EDGES, 128), jnp.float32),
        ],
        interpret=interpret,
    )(gs, gd, ve, w1s, w1d, w1e, b1, w2f, b2f, w2k, b2k, wvf, wve, bv)


def _sc_sums(f1, f4, dst, zrows, ones128):
    CH = 80
    per_w = N_EDGES // 16
    HN = N_NODES // 2
    AR = HN + 16
    ZR = 200
    NZC = HN // ZR

    @functools.partial(
        pl.kernel,
        mesh=plsc.VectorSubcoreMesh(**_MESH),
        compiler_params=pltpu.CompilerParams(needs_layout_passes=False),
        out_type=[
            jax.ShapeDtypeStruct((N_NODES, 128), jnp.float32),
            jax.ShapeDtypeStruct((N_NODES, 128), jnp.float32),
            jax.ShapeDtypeStruct((N_NODES, 128), jnp.float32),
        ],
        scratch_types=[
            pltpu.VMEM((CH,), jnp.int32),
            pltpu.VMEM((CH,), jnp.int32),
            pltpu.VMEM((CH, 128), jnp.float32),
            pltpu.VMEM((CH, 128), jnp.float32),
            pltpu.VMEM_SHARED((AR, 128), jnp.float32),
        ],
    )
    def k(f1_hbm, f4_hbm, dst_hbm, z_hbm, o_hbm, nf1_hbm, nf4_hbm, deg_hbm,
          dbuf, dbuf2, fbuf, obuf, acc_sh):
        core = lax.axis_index("c")
        s = lax.axis_index("s")
        nzc = (NZC - 1 - s) // 16 + 1
        pltpu.sync_copy(o_hbm, obuf)

        def zero_acc():
            def zinit(q, _):
                r0 = (s + q * 16) * ZR
                pltpu.sync_copy(z_hbm, acc_sh.at[pl.ds(r0, ZR)])
                return ()

            lax.fori_loop(0, nzc, zinit, ())

            @pl.when(s == 0)
            def _():
                pltpu.sync_copy(z_hbm.at[pl.ds(0, 16)], acc_sh.at[pl.ds(HN, 16)])

        def remap(base):
            def sub(j, _):
                dv = dbuf[pl.ds(j * 16, 16)]
                inr = (dv >= base) & (dv < base + HN)
                tr = HN + jnp.bitwise_and(dv, 15)
                dbuf2[pl.ds(j * 16, 16)] = jnp.where(inr, dv - base, tr)
                return ()
            lax.fori_loop(0, CH // 16, sub, ())

        def writeout(dst_out, base):
            def wo(q, _):
                r0 = (s + q * 16) * ZR
                pltpu.sync_copy(acc_sh.at[pl.ds(r0, ZR)],
                                dst_out.at[pl.ds(base + r0, ZR)])
                return ()
            lax.fori_loop(0, nzc, wo, ())

        def f_pass(p):
            base = p * HN
            zero_acc()
            plsc.subcore_barrier()

            def step(i, _):
                eb = s * per_w + i * CH
                pltpu.sync_copy(dst_hbm.at[pl.ds(eb, CH)], dbuf)
                remap(base)

                @pl.when(core == 0)
                def _():
                    pltpu.sync_copy(f1_hbm.at[pl.ds(eb, CH)], fbuf)
                    pltpu.sync_copy(fbuf, acc_sh.at[dbuf2], add=True)

                @pl.when(core == 1)
                def _():
                    pltpu.sync_copy(f4_hbm.at[pl.ds(eb, CH)], fbuf)
                    pltpu.sync_copy(fbuf, acc_sh.at[dbuf2], add=True)
                return ()

            lax.fori_loop(0, per_w // CH, step, ())
            plsc.subcore_barrier()

            @pl.when(core == 0)
            def _():
                writeout(nf1_hbm, base)

            @pl.when(core == 1)
            def _():
                writeout(nf4_hbm, base)

            plsc.subcore_barrier()

        def deg_pass(p):
            base = p * HN
            zero_acc()
            plsc.subcore_barrier()

            def step(i, _):
                eb = s * per_w + i * CH
                pltpu.sync_copy(dst_hbm.at[pl.ds(eb, CH)], dbuf)
                remap(base)
                pltpu.sync_copy(obuf, acc_sh.at[dbuf2], add=True)
                return ()

            @pl.when(core == 1)
            def _():
                lax.fori_loop(0, per_w // CH, step, ())

            plsc.subcore_barrier()

            @pl.when(core == 1)
            def _():
                writeout(deg_hbm, base)

            plsc.subcore_barrier()

        f_pass(0)
        f_pass(1)
        deg_pass(0)
        deg_pass(1)

    return k(f1, f4, dst, zrows, ones128)


def _sc_maxmin(fmm_t, dst):
    CH = 640
    NG = CH // 16
    TMPN = 4096

    @functools.partial(
        pl.kernel,
        mesh=plsc.VectorSubcoreMesh(**_MESH),
        compiler_params=pltpu.CompilerParams(needs_layout_passes=False),
        out_type=[jax.ShapeDtypeStruct((256, N_NODES), jnp.float32)],
        scratch_types=[
            pltpu.VMEM((CH,), jnp.int32),
            pltpu.VMEM((4, CH), jnp.float32),
            pltpu.VMEM((4, N_NODES), jnp.float32),
            pltpu.VMEM((TMPN,), jnp.int32),
        ],
    )
    def k(fmm_hbm, dst_hbm, out_hbm, dbuf, fbuf, acc, tmp):
        w = _wid()
        ninf = jnp.full((16,), -jnp.inf, jnp.float32)
        iota = lax.iota(jnp.int32, 16)
        csplat = [jnp.full((16,), c, jnp.int32) for c in range(4)]

        def one_round(r):
            sgn = jnp.where(r == 0, 1.0, -1.0).astype(jnp.float32)
            sgnv = jnp.full((16,), 1.0, jnp.float32) * sgn
            c0 = (w + r * 32) * 4

            def ini(j, _):
                acc[j // (N_NODES // 16),
                    pl.ds((j % (N_NODES // 16)) * 16, 16)] = ninf
                return ()

            lax.fori_loop(0, 4 * (N_NODES // 16), ini, ())

            def rmw(d, rowv, mask):
                for c in range(4):
                    vc = plsc.load_gather(fbuf, [csplat[c], rowv]) * sgnv
                    a = plsc.load_gather(acc, [csplat[c], d])
                    na = jnp.maximum(a, vc)
                    if mask is None:
                        plsc.store_scatter(acc, [csplat[c], d], na)
                    else:
                        plsc.store_scatter(acc, [csplat[c], d], na, mask=mask)

            def group(g, _):
                d = dbuf[pl.ds(g * 16, 16)]
                rowv = iota + g * 16
                dh = jnp.bitwise_and(d, TMPN - 1)
                plsc.store_scatter(tmp, [dh], iota)
                rr = plsc.load_gather(tmp, [dh])
                ndup = jnp.max(jnp.where(rr != iota, 1, 0))

                @pl.when(ndup == 0)
                def _():
                    rmw(d, rowv, None)

                @pl.when(ndup != 0)
                def _():
                    def lane(l, _):
                        rmw(d, rowv, iota == l)
                        return ()
                    lax.fori_loop(0, 16, lane, ())
                return ()

            def chunk(i, _):
                e0 = i * CH
                pltpu.sync_copy(dst_hbm.at[pl.ds(e0, CH)], dbuf)
                pltpu.sync_copy(fmm_hbm.at[pl.ds(c0, 4), pl.ds(e0, CH)], fbuf)
                lax.fori_loop(0, NG, group, ())
                return ()

            lax.fori_loop(0, N_EDGES // CH, chunk, ())
            pltpu.sync_copy(acc, out_hbm.at[pl.ds(c0, 4)])

        one_round(0)
        one_round(1)

    return k(fmm_t, dst)


def _node_head_body(vc, nf1, nfmm, nf4, degx, wr, br, out):
    d = degx[...][:, :1]
    has = d > 0.0
    nf2 = jnp.where(has, nfmm[...][:, :128], 0.0)
    nf3 = jnp.where(has, -nfmm[...][:, 128:], 0.0)
    nf4v = nf4[...] / jnp.maximum(d, 1.0)
    x = jnp.concatenate([vc[...], nf1[...], nf2, nf3, nf4v], axis=1)
    out[...] = jnp.dot(x.astype(jnp.bfloat16), wr[...],
                       preferred_element_type=jnp.float32) + br[...]


def _node_head(in_vc, nf1, nfmm, nf4, degx, wr, br, interpret=False):
    NB = 2000
    grid = (N_NODES // NB,)
    nbs = lambda cols: pl.BlockSpec((NB, cols), lambda i: (i, 0))
    full = lambda a: pl.BlockSpec(a.shape, lambda i: (0,) * a.ndim)
    return pl.pallas_call(
        _node_head_body,
        grid=grid,
        in_specs=[nbs(128), nbs(128), nbs(256), nbs(128), nbs(128),
                  full(wr), full(br)],
        out_specs=nbs(128),
        out_shape=jax.ShapeDtypeStruct((N_NODES, 128), jnp.float32),
        interpret=interpret,
    )(in_vc, nf1, nfmm, nf4, degx, wr, br)


def kernel(in_vc, in_ve, edge_index, W1, b1, W2, b2, Wr, br, Wv, bv):
    bf16 = jnp.bfloat16
    src = edge_index[0]
    dst = edge_index[1]
    w1s = W1[:128].astype(bf16)
    w1d = W1[128:256].astype(bf16)
    w1e = W1[256:].astype(bf16)
    b1r = b1.reshape(1, 384)
    w2f = W2[:, 1:513].astype(bf16)
    b2f = b2[1:513].reshape(1, 512)
    w2k = W2[:, 0].reshape(1, 384)
    b2k = b2[0].reshape(1, 1)
    wvf = Wv[:512].astype(bf16)
    wve = Wv[512:].astype(bf16)
    bvr = bv.reshape(1, 128)
    wr16 = Wr.astype(bf16)
    brr = br.reshape(1, 128)

    gs, gd = _sc_gather(in_vc, src, dst)
    f1, f4, fmm, out_ve = _edge_mlp(gs, gd, in_ve, w1s, w1d, w1e, b1r, w2f,
                                    b2f, w2k, b2k, wvf, wve, bvr)
    zrows = jnp.zeros((200, 128), jnp.float32)
    ones128 = jnp.ones((80, 128), jnp.float32)
    nf1, nf4, degx = _sc_sums(f1, f4, dst, zrows, ones128)
    (nfmm_t,) = _sc_maxmin(jnp.transpose(fmm), dst)
    nfmm = jnp.transpose(nfmm_t)
    out_vc = _node_head(in_vc, nf1, nfmm, nf4, degx, wr16, brr)
    return (out_vc, out_ve)


# speedup vs baseline: 1.0566x; 1.0566x over previous
import functools

import jax
import jax.numpy as jnp
from jax import lax
from jax.experimental import pallas as pl
from jax.experimental.pallas import tpu as pltpu
from jax.experimental.pallas import tpu_sc as plsc

N_NODES = 10000
N_EDGES = 320000
NW = 32

_MESH = dict(core_axis_name="c", subcore_axis_name="s")


def _wid():
    return lax.axis_index("s") * 2 + lax.axis_index("c")


def _sc_gather(in_vc, src, dst):
    CH = 80
    per_w = N_EDGES // NW

    @functools.partial(
        pl.kernel,
        mesh=plsc.VectorSubcoreMesh(**_MESH),
        compiler_params=pltpu.CompilerParams(needs_layout_passes=False),
        out_type=[
            jax.ShapeDtypeStruct((N_EDGES, 128), jnp.float32),
            jax.ShapeDtypeStruct((N_EDGES, 128), jnp.float32),
        ],
        scratch_types=[
            pltpu.VMEM((CH,), jnp.int32),
            pltpu.VMEM((CH, 128), jnp.float32),
            pltpu.SemaphoreType.DMA,
        ],
    )
    def k(vc_hbm, src_hbm, dst_hbm, gs_hbm, gd_hbm, idx_v, rows_v, sem):
        w = _wid()
        base0 = w * per_w

        def step(i, _):
            base = base0 + i * CH
            pltpu.sync_copy(src_hbm.at[pl.ds(base, CH)], idx_v)
            pltpu.async_copy(vc_hbm.at[idx_v], rows_v, sem).wait()
            pltpu.sync_copy(rows_v, gs_hbm.at[pl.ds(base, CH)])
            pltpu.sync_copy(dst_hbm.at[pl.ds(base, CH)], idx_v)
            pltpu.async_copy(vc_hbm.at[idx_v], rows_v, sem).wait()
            pltpu.sync_copy(rows_v, gd_hbm.at[pl.ds(base, CH)])
            return ()

        lax.fori_loop(0, per_w // CH, step, ())

    return k(in_vc, src, dst)


def _edge_mlp_body(gs, gd, ve, w1s, w1d, w1e, b1, w2f, b2f, w2k, b2k,
                   wvf, wve, bv, f1o, f4o, fmmo, oveo):
    f32 = jnp.float32
    gs16 = gs[...].astype(jnp.bfloat16)
    gd16 = gd[...].astype(jnp.bfloat16)
    ve16 = ve[...].astype(jnp.bfloat16)
    h1 = (jnp.dot(gs16, w1s[...], preferred_element_type=f32)
          + jnp.dot(gd16, w1d[...], preferred_element_type=f32)
          + jnp.dot(ve16, w1e[...], preferred_element_type=f32) + b1[...])
    h1 = jnp.maximum(h1, 0.0)
    h1b = h1.astype(jnp.bfloat16)
    hf = jnp.dot(h1b, w2f[...], preferred_element_type=f32) + b2f[...]
    hk = jnp.sum(h1 * w2k[...], axis=1, keepdims=True) + b2k[...]
    kgate = jax.nn.sigmoid(hk)
    f = hf * kgate
    f1o[...] = f[:, :128]
    f4o[...] = f[:, 384:]
    fmmo[...] = f[:, 128:384]
    oveo[...] = (jnp.dot(f.astype(jnp.bfloat16), wvf[...], preferred_element_type=f32)
                 + jnp.dot(ve16, wve[...], preferred_element_type=f32) + bv[...])


def _edge_mlp(gs, gd, ve, w1s, w1d, w1e, b1, w2f, b2f, w2k, b2k, wvf, wve, bv,
              interpret=False):
    EB = 2560
    grid = (N_EDGES // EB,)
    ebs = lambda cols: pl.BlockSpec((EB, cols), lambda i: (i, 0))
    full = lambda a: pl.BlockSpec(a.shape, lambda i: (0,) * a.ndim)
    return pl.pallas_call(
        _edge_mlp_body,
        grid=grid,
        in_specs=[ebs(128), ebs(128), ebs(128), full(w1s), full(w1d),
                  full(w1e), full(b1), full(w2f), full(b2f), full(w2k),
                  full(b2k), full(wvf), full(wve), full(bv)],
        out_specs=[ebs(128), ebs(128), ebs(256), ebs(128)],
        out_shape=[
            jax.ShapeDtypeStruct((N_EDGES, 128), jnp.float32),
            jax.ShapeDtypeStruct((N_EDGES, 128), jnp.float32),
            jax.ShapeDtypeStruct((N_EDGES, 256), jnp.float32),
            jax.ShapeDtypeStruct((N_EDGES, 128), jnp.float32),
        ],
        interpret=interpret,
    )(gs, gd, ve, w1s, w1d, w1e, b1, w2f, b2f, w2k, b2k, wvf, wve, bv)


def _sc_sums(f1, f4, dst, zrows, ones128):
    CH = 160
    per_w = N_EDGES // 16
    HN = N_NODES // 2
    AR = HN + 16
    ZR = 200
    NZC = HN // ZR

    @functools.partial(
        pl.kernel,
        mesh=plsc.VectorSubcoreMesh(**_MESH),
        compiler_params=pltpu.CompilerParams(needs_layout_passes=False),
        out_type=[
            jax.ShapeDtypeStruct((N_NODES, 128), jnp.float32),
            jax.ShapeDtypeStruct((N_NODES, 128), jnp.float32),
            jax.ShapeDtypeStruct((N_NODES, 128), jnp.float32),
        ],
        scratch_types=[
            pltpu.VMEM((CH,), jnp.int32),
            pltpu.VMEM((CH,), jnp.int32),
            pltpu.VMEM((CH, 128), jnp.float32),
            pltpu.VMEM((80, 128), jnp.float32),
            pltpu.VMEM_SHARED((AR, 128), jnp.float32),
        ],
    )
    def k(f1_hbm, f4_hbm, dst_hbm, z_hbm, o_hbm, nf1_hbm, nf4_hbm, deg_hbm,
          dbuf, dbuf2, fbuf, obuf, acc_sh):
        core = lax.axis_index("c")
        s = lax.axis_index("s")
        nzc = (NZC - 1 - s) // 16 + 1
        pltpu.sync_copy(o_hbm, obuf)

        def zero_acc():
            def zinit(q, _):
                r0 = (s + q * 16) * ZR
                pltpu.sync_copy(z_hbm, acc_sh.at[pl.ds(r0, ZR)])
                return ()

            lax.fori_loop(0, nzc, zinit, ())

            @pl.when(s == 0)
            def _():
                pltpu.sync_copy(z_hbm.at[pl.ds(0, 16)], acc_sh.at[pl.ds(HN, 16)])

        def remap(base):
            def sub(j, _):
                dv = dbuf[pl.ds(j * 16, 16)]
                inr = (dv >= base) & (dv < base + HN)
                tr = HN + jnp.bitwise_and(dv, 15)
                dbuf2[pl.ds(j * 16, 16)] = jnp.where(inr, dv - base, tr)
                return ()
            lax.fori_loop(0, CH // 16, sub, ())

        def writeout(dst_out, base):
            def wo(q, _):
                r0 = (s + q * 16) * ZR
                pltpu.sync_copy(acc_sh.at[pl.ds(r0, ZR)],
                                dst_out.at[pl.ds(base + r0, ZR)])
                return ()
            lax.fori_loop(0, nzc, wo, ())

        def f_pass(p):
            base = p * HN
            zero_acc()
            plsc.subcore_barrier()

            def step(i, _):
                eb = s * per_w + i * CH
                pltpu.sync_copy(dst_hbm.at[pl.ds(eb, CH)], dbuf)
                remap(base)

                @pl.when(core == 0)
                def _():
                    pltpu.sync_copy(f1_hbm.at[pl.ds(eb, CH)], fbuf)
                    pltpu.sync_copy(fbuf, acc_sh.at[dbuf2], add=True)

                @pl.when(core == 1)
                def _():
                    pltpu.sync_copy(f4_hbm.at[pl.ds(eb, CH)], fbuf)
                    pltpu.sync_copy(fbuf, acc_sh.at[dbuf2], add=True)
                return ()

            lax.fori_loop(0, per_w // CH, step, ())
            plsc.subcore_barrier()

            @pl.when(core == 0)
            def _():
                writeout(nf1_hbm, base)

            @pl.when(core == 1)
            def _():
                writeout(nf4_hbm, base)

            plsc.subcore_barrier()

        def deg_pass(p):
            base = p * HN
            zero_acc()
            plsc.subcore_barrier()

            def step(i, _):
                eb = s * per_w + i * CH
                pltpu.sync_copy(dst_hbm.at[pl.ds(eb, CH)], dbuf)
                remap(base)
                pltpu.sync_copy(obuf, acc_sh.at[dbuf2.at[pl.ds(0, 80)]], add=True)
                pltpu.sync_copy(obuf, acc_sh.at[dbuf2.at[pl.ds(80, 80)]], add=True)
                return ()

            @pl.when(core == 1)
            def _():
                lax.fori_loop(0, per_w // CH, step, ())

            plsc.subcore_barrier()

            @pl.when(core == 1)
            def _():
                writeout(deg_hbm, base)

            plsc.subcore_barrier()

        f_pass(0)
        f_pass(1)
        deg_pass(0)
        deg_pass(1)

    return k(f1, f4, dst, zrows, ones128)


def _sc_maxmin(fmm_t, dst):
    CH = 640
    NG = CH // 16
    TMPN = 4096

    @functools.partial(
        pl.kernel,
        mesh=plsc.VectorSubcoreMesh(**_MESH),
        compiler_params=pltpu.CompilerParams(needs_layout_passes=False),
        out_type=[jax.ShapeDtypeStruct((256, N_NODES), jnp.float32)],
        scratch_types=[
            pltpu.VMEM((CH,), jnp.int32),
            pltpu.VMEM((4, CH), jnp.float32),
            pltpu.VMEM((4, N_NODES), jnp.float32),
            pltpu.VMEM((TMPN,), jnp.int32),
        ],
    )
    def k(fmm_hbm, dst_hbm, out_hbm, dbuf, fbuf, acc, tmp):
        w = _wid()
        ninf = jnp.full((16,), -jnp.inf, jnp.float32)
        iota = lax.iota(jnp.int32, 16)
        csplat = [jnp.full((16,), c, jnp.int32) for c in range(4)]

        def one_round(r):
            sgn = jnp.where(r == 0, 1.0, -1.0).astype(jnp.float32)
            sgnv = jnp.full((16,), 1.0, jnp.float32) * sgn
            c0 = (w + r * 32) * 4

            def ini(j, _):
                acc[j // (N_NODES // 16),
                    pl.ds((j % (N_NODES // 16)) * 16, 16)] = ninf
                return ()

            lax.fori_loop(0, 4 * (N_NODES // 16), ini, ())

            def rmw(d, rowv, mask):
                for c in range(4):
                    vc = plsc.load_gather(fbuf, [csplat[c], rowv]) * sgnv
                    a = plsc.load_gather(acc, [csplat[c], d])
                    na = jnp.maximum(a, vc)
                    if mask is None:
                        plsc.store_scatter(acc, [csplat[c], d], na)
                    else:
                        plsc.store_scatter(acc, [csplat[c], d], na, mask=mask)

            def group(g, _):
                d = dbuf[pl.ds(g * 16, 16)]
                rowv = iota + g * 16
                dh = jnp.bitwise_and(d, TMPN - 1)
                plsc.store_scatter(tmp, [dh], iota)
                rr = plsc.load_gather(tmp, [dh])
                ndup = jnp.max(jnp.where(rr != iota, 1, 0))

                @pl.when(ndup == 0)
                def _():
                    rmw(d, rowv, None)

                @pl.when(ndup != 0)
                def _():
                    def lane(l, _):
                        rmw(d, rowv, iota == l)
                        return ()
                    lax.fori_loop(0, 16, lane, ())
                return ()

            def chunk(i, _):
                e0 = i * CH
                pltpu.sync_copy(dst_hbm.at[pl.ds(e0, CH)], dbuf)
                pltpu.sync_copy(fmm_hbm.at[pl.ds(c0, 4), pl.ds(e0, CH)], fbuf)
                lax.fori_loop(0, NG, group, ())
                return ()

            lax.fori_loop(0, N_EDGES // CH, chunk, ())
            pltpu.sync_copy(acc, out_hbm.at[pl.ds(c0, 4)])

        one_round(0)
        one_round(1)

    return k(fmm_t, dst)


def _node_head_body(vc, nf1, nfmm, nf4, degx, wr, br, out):
    d = degx[...][:, :1]
    has = d > 0.0
    nf2 = jnp.where(has, nfmm[...][:, :128], 0.0)
    nf3 = jnp.where(has, -nfmm[...][:, 128:], 0.0)
    nf4v = nf4[...] / jnp.maximum(d, 1.0)
    x = jnp.concatenate([vc[...], nf1[...], nf2, nf3, nf4v], axis=1)
    out[...] = jnp.dot(x.astype(jnp.bfloat16), wr[...],
                       preferred_element_type=jnp.float32) + br[...]


def _node_head(in_vc, nf1, nfmm, nf4, degx, wr, br, interpret=False):
    NB = 2000
    grid = (N_NODES // NB,)
    nbs = lambda cols: pl.BlockSpec((NB, cols), lambda i: (i, 0))
    full = lambda a: pl.BlockSpec(a.shape, lambda i: (0,) * a.ndim)
    return pl.pallas_call(
        _node_head_body,
        grid=grid,
        in_specs=[nbs(128), nbs(128), nbs(256), nbs(128), nbs(128),
                  full(wr), full(br)],
        out_specs=nbs(128),
        out_shape=jax.ShapeDtypeStruct((N_NODES, 128), jnp.float32),
        interpret=interpret,
    )(in_vc, nf1, nfmm, nf4, degx, wr, br)


def kernel(in_vc, in_ve, edge_index, W1, b1, W2, b2, Wr, br, Wv, bv):
    bf16 = jnp.bfloat16
    src = edge_index[0]
    dst = edge_index[1]
    w1s = W1[:128].astype(bf16)
    w1d = W1[128:256].astype(bf16)
    w1e = W1[256:].astype(bf16)
    b1r = b1.reshape(1, 384)
    w2f = W2[:, 1:513].astype(bf16)
    b2f = b2[1:513].reshape(1, 512)
    w2k = W2[:, 0].reshape(1, 384)
    b2k = b2[0].reshape(1, 1)
    wvf = Wv[:512].astype(bf16)
    wve = Wv[512:].astype(bf16)
    bvr = bv.reshape(1, 128)
    wr16 = Wr.astype(bf16)
    brr = br.reshape(1, 128)

    gs, gd = _sc_gather(in_vc, src, dst)
    f1, f4, fmm, out_ve = _edge_mlp(gs, gd, in_ve, w1s, w1d, w1e, b1r, w2f,
                                    b2f, w2k, b2k, wvf, wve, bvr)
    zrows = jnp.zeros((200, 128), jnp.float32)
    ones128 = jnp.ones((80, 128), jnp.float32)
    nf1, nf4, degx = _sc_sums(f1, f4, dst, zrows, ones128)
    (nfmm_t,) = _sc_maxmin(jnp.transpose(fmm), dst)
    nfmm = jnp.transpose(nfmm_t)
    out_vc = _node_head(in_vc, nf1, nfmm, nf4, degx, wr16, brr)
    return (out_vc, out_ve)
